# baseline (device time: 69266 ns/iter reference)
import jax
import jax.numpy as jnp
from jax import lax
from jax.experimental import pallas as pl
from jax.experimental.pallas import tpu as pltpu

N_DEV = 4
SQ = 1024
D = 1024
HQ_PER = 8
DH = 128
BLK = 64
SCALE = 0.08838834764831843
SUB = 128
NSUB = SQ // SUB

_MESH = pl.DeviceIdType.MESH


def kernel(x, Wq, K_ext, V_ext, Wo):
    i = lax.axis_index("i")
    x2 = x.reshape(SQ, D)
    k_my = lax.dynamic_slice(
        K_ext, (0, 0, i * HQ_PER, 0), (1, SQ, HQ_PER, DH)
    ).reshape(SQ, HQ_PER * DH)
    v_my = lax.dynamic_slice(
        V_ext, (0, 0, i * HQ_PER, 0), (1, SQ, HQ_PER, DH)
    ).reshape(SQ, HQ_PER * DH)

    def body(
        x_ref, wq_ref, k_ref, v_ref, wo_ref, out_ref,
        qc_ref, ctxt_ref, own_ref, rs_send_ref, rs_recv_ref,
        ag_send_ref, ag_recv_ref,
        rs_send_sems, rs_recv_sems, ag_send_sems, ag_recv_sems,
    ):
        my = lax.axis_index("i")

        barrier_sem = pltpu.get_barrier_semaphore()
        for o in range(1, N_DEV):
            peer = lax.rem(my + o, N_DEV)
            pl.semaphore_signal(
                barrier_sem, inc=1, device_id=(peer,), device_id_type=_MESH
            )
        pl.semaphore_wait(barrier_sem, N_DEV - 1)

        xb = x_ref[...].astype(jnp.bfloat16)
        wqb = wq_ref[...].astype(jnp.bfloat16)
        q = lax.dot_general(
            xb, wqb, (((1,), (0,)), ((), ())),
            preferred_element_type=jnp.float32,
        )
        qc_ref[...] = (q * SCALE).astype(jnp.bfloat16)

        kb = k_ref[...].astype(jnp.bfloat16)
        vb = v_ref[...].astype(jnp.bfloat16)
        wob = wo_ref[...].astype(jnp.bfloat16)

        def reduce_and_ag(c):
            par = c % 2
            for rel in range(1, N_DEV):
                slot = par * N_DEV + rel
                pltpu.make_async_remote_copy(
                    src_ref=rs_recv_ref.at[slot],
                    dst_ref=rs_recv_ref.at[slot],
                    send_sem=rs_send_sems.at[0],
                    recv_sem=rs_recv_sems.at[slot],
                    device_id=(my,),
                    device_id_type=_MESH,
                ).wait_recv()
            red = (
                own_ref[par, :, :]
                + rs_recv_ref[par * N_DEV + 1, :, :].astype(jnp.float32)
                + rs_recv_ref[par * N_DEV + 2, :, :].astype(jnp.float32)
                + rs_recv_ref[par * N_DEV + 3, :, :].astype(jnp.float32)
            )
            out_ref[c * SUB:(c + 1) * SUB, :] = red
            ag_send_ref[par, :, :] = red.astype(jnp.bfloat16)
            for o in range(1, N_DEV):
                peer = lax.rem(my + o, N_DEV)
                pltpu.make_async_remote_copy(
                    src_ref=ag_send_ref.at[par],
                    dst_ref=ag_recv_ref.at[c],
                    send_sem=ag_send_sems.at[par * N_DEV + o],
                    recv_sem=ag_recv_sems.at[c],
                    device_id=(peer,),
                    device_id_type=_MESH,
                ).start()

        for s in range(NSUB):
            L = SUB * (s + 1)
            r0 = s * SUB
            own = s // 2
            rowb = (r0 + lax.broadcasted_iota(jnp.int32, (SUB, L), 0)) // BLK
            colb = lax.broadcasted_iota(jnp.int32, (SUB, L), 1) // BLK
            neg_t = jnp.where(colb <= rowb, 0.0, -1e9).astype(jnp.float32)

            for h in range(HQ_PER):
                c0 = h * DH
                qh = qc_ref[r0:r0 + SUB, c0:c0 + DH]
                kh = kb[:L, c0:c0 + DH]
                sc = lax.dot_general(
                    qh, kh, (((1,), (1,)), ((), ())),
                    preferred_element_type=jnp.float32,
                )
                w = jnp.exp(sc + neg_t)
                denom = jnp.sum(w, axis=1, keepdims=True)
                p = w.astype(jnp.bfloat16)
                ctx = lax.dot_general(
                    p, vb[:L, c0:c0 + DH], (((1,), (0,)), ((), ())),
                    preferred_element_type=jnp.float32,
                )
                ctxt_ref[:, c0:c0 + DH] = (ctx / denom).astype(jnp.bfloat16)

            partial = lax.dot_general(
                ctxt_ref[...], wob, (((1,), (0,)), ((), ())),
                preferred_element_type=jnp.float32,
            )
            rs_send_ref[s, :, :] = partial.astype(jnp.bfloat16)

            @pl.when(my == own)
            def _():
                own_ref[s % 2, :, :] = partial

            @pl.when(my != own)
            def _():
                rel = lax.rem(own - my + N_DEV, N_DEV)
                pltpu.make_async_remote_copy(
                    src_ref=rs_send_ref.at[s],
                    dst_ref=rs_recv_ref.at[(s % 2) * N_DEV + rel],
                    send_sem=rs_send_sems.at[s],
                    recv_sem=rs_recv_sems.at[(s % 2) * N_DEV + rel],
                    device_id=(own,),
                    device_id_type=_MESH,
                ).start()

            if s >= 2:
                c = s - 2

                @pl.when(my == c // 2)
                def _():
                    reduce_and_ag(c)

        for c in (NSUB - 2, NSUB - 1):
            @pl.when(my == c // 2)
            def _():
                reduce_and_ag(c)

        for c in range(NSUB):
            @pl.when(my != c // 2)
            def _():
                pltpu.make_async_remote_copy(
                    src_ref=ag_recv_ref.at[c],
                    dst_ref=ag_recv_ref.at[c],
                    send_sem=ag_send_sems.at[0],
                    recv_sem=ag_recv_sems.at[c],
                    device_id=(my,),
                    device_id_type=_MESH,
                ).wait_recv()
                out_ref[c * SUB:(c + 1) * SUB, :] = (
                    ag_recv_ref[c, :, :].astype(jnp.float32)
                )

        for s in range(NSUB):
            @pl.when(my != s // 2)
            def _():
                pltpu.make_async_remote_copy(
                    src_ref=rs_send_ref.at[s],
                    dst_ref=rs_recv_ref.at[0],
                    send_sem=rs_send_sems.at[s],
                    recv_sem=rs_recv_sems.at[0],
                    device_id=(my,),
                    device_id_type=_MESH,
                ).wait_send()
        for par in range(2):
            for o in range(1, N_DEV):
                pltpu.make_async_remote_copy(
                    src_ref=ag_send_ref.at[par],
                    dst_ref=ag_recv_ref.at[0],
                    send_sem=ag_send_sems.at[par * N_DEV + o],
                    recv_sem=ag_recv_sems.at[0],
                    device_id=(my,),
                    device_id_type=_MESH,
                ).wait_send()

    out = pl.pallas_call(
        body,
        out_shape=jax.ShapeDtypeStruct((SQ, D), jnp.float32),
        in_specs=[pl.BlockSpec(memory_space=pltpu.VMEM)] * 5,
        out_specs=pl.BlockSpec(memory_space=pltpu.VMEM),
        scratch_shapes=[
            pltpu.VMEM((SQ, D), jnp.bfloat16),
            pltpu.VMEM((SUB, D), jnp.bfloat16),
            pltpu.VMEM((2, SUB, D), jnp.float32),
            pltpu.VMEM((NSUB, SUB, D), jnp.bfloat16),
            pltpu.VMEM((2 * N_DEV, SUB, D), jnp.bfloat16),
            pltpu.VMEM((2, SUB, D), jnp.bfloat16),
            pltpu.VMEM((NSUB, SUB, D), jnp.bfloat16),
            pltpu.SemaphoreType.DMA((NSUB,)),
            pltpu.SemaphoreType.DMA((2 * N_DEV,)),
            pltpu.SemaphoreType.DMA((2 * N_DEV,)),
            pltpu.SemaphoreType.DMA((NSUB,)),
        ],
        compiler_params=pltpu.CompilerParams(collective_id=0),
    )(x2, Wq, k_my, v_my, Wo)
    return out.reshape(1, SQ, D)
